# baseline (speedup 1.0000x reference)
import functools

import jax
import jax.numpy as jnp
from jax import lax
from jax.experimental import pallas as pl
from jax.experimental.pallas import tpu as pltpu
from jax.experimental.pallas import tpu_sc as plsc

B, L = 4096, 64
HIDDEN, OUT_DIM = 128, 256
LANES = 16
H_REGS = HIDDEN // LANES

NUM_CORES = 2
NUM_SUBCORES = 16
NW = NUM_CORES * NUM_SUBCORES
B_PER_W = B // NW

IDS_PER_CHUNK = 128
TOK_PER_W = B_PER_W * L
GROUPS = L // LANES
MAX_CHUNKS = B_PER_W + 1
CSTREAM = MAX_CHUNKS * IDS_PER_CHUNK + 2 * LANES


def _sc_pool_build():
    mesh = plsc.VectorSubcoreMesh(core_axis_name="c", subcore_axis_name="s")

    @functools.partial(
        pl.kernel,
        mesh=mesh,
        out_type=jax.ShapeDtypeStruct((B, HIDDEN), jnp.float32),
        scratch_types=[
            pltpu.VMEM((B_PER_W, L), jnp.int32),
            pltpu.VMEM((B_PER_W, L), jnp.float32),
            pltpu.VMEM((CSTREAM,), jnp.int32),
            pltpu.VMEM((IDS_PER_CHUNK, HIDDEN), jnp.float32),
            pltpu.VMEM((IDS_PER_CHUNK, HIDDEN), jnp.float32),
            pltpu.VMEM((B_PER_W, HIDDEN), jnp.float32),
            pltpu.SMEM((B_PER_W,), jnp.int32),
            pltpu.SMEM((B_PER_W,), jnp.int32),
            pltpu.SMEM((MAX_CHUNKS + 2,), jnp.int32),
            pltpu.SemaphoreType.DMA,
            pltpu.SemaphoreType.DMA,
        ],
        compiler_params=pltpu.CompilerParams(needs_layout_passes=False),
    )
    def sc_pool(ids_hbm, mask_hbm, table_hbm, out_hbm,
                idx_v, mask_v, cidx_v, rows0, rows1, pooled_v,
                cnt_smem, rstart_smem, nrows_smem, sem0, sem1):
        wid = lax.axis_index("s") * NUM_CORES + lax.axis_index("c")
        row_base = wid * B_PER_W

        pltpu.sync_copy(ids_hbm.at[pl.ds(row_base, B_PER_W)], idx_v)
        pltpu.sync_copy(mask_hbm.at[pl.ds(row_base, B_PER_W)], mask_v)

        def zero_nrows(c, _):
            nrows_smem[c] = jnp.int32(0)
            return 0

        lax.fori_loop(0, MAX_CHUNKS + 2, zero_nrows, 0)

        pad16 = lax.iota(jnp.int32, LANES)
        pad_salt = (wid * jnp.int32(1009)) & jnp.int32(65535)

        def pad_to_bin(pos):
            pos_new = (pos + (IDS_PER_CHUNK - 1)) & ~(IDS_PER_CHUNK - 1)
            kpad = pos_new - pos

            def store_pad(j, _):
                p0 = pos + j * LANES
                base = (p0 * jnp.int32(37) + pad_salt) & jnp.int32(65535)
                cidx_v[pl.ds(p0, LANES)] = (
                    jnp.full((LANES,), base, jnp.int32) + pad16)
                return 0

            lax.fori_loop(0, (kpad + LANES - 1) >> 4, store_pad, 0)
            return pos_new

        def compact(row, pos):
            idgs, mgs, pcnts = [], [], []
            c = jnp.int32(0)
            for g in range(GROUPS):
                o = g * LANES
                idg = idx_v[row, pl.ds(o, LANES)]
                mg = mask_v[row, pl.ds(o, LANES)] > 0.0
                pc = plsc.all_reduce_population_count(mg)[0]
                idgs.append(idg)
                mgs.append(mg)
                pcnts.append(pc)
                c = c + pc
            rem = pos & (IDS_PER_CHUNK - 1)
            needs_new_bin = rem + c > IDS_PER_CHUNK
            pos = lax.cond(needs_new_bin, pad_to_bin, lambda p: p, pos)
            cnt_smem[row] = c
            rstart_smem[row] = pos
            b = lax.shift_right_logical(pos, 7)
            nrows_smem[b] = nrows_smem[b] + 1
            for g in range(GROUPS):
                plsc.store_compressed(cidx_v.at[pl.ds(pos, LANES)],
                                      idgs[g], mask=mgs[g])
                pos = pos + pcnts[g]
            return pos

        def compact_until(row_c, pos, target):
            def cond(st):
                r, p = st
                return jnp.logical_and(r < B_PER_W,
                                       p < (target + 1) * IDS_PER_CHUNK)

            def body(st):
                r, p = st
                return (r + 1, compact(r, p))

            row_c, pos = lax.while_loop(cond, body, (row_c, pos))
            pos = lax.cond(row_c == B_PER_W, pad_to_bin, lambda p: p, pos)
            return row_c, pos

        def gather_start(chunk, buf, sem):
            pltpu.make_async_copy(
                table_hbm.at[cidx_v.at[pl.ds(chunk * IDS_PER_CHUNK,
                                             IDS_PER_CHUNK)]],
                buf, sem).start()

        def gather_wait(buf, sem):
            pltpu.make_async_copy(
                table_hbm.at[cidx_v.at[pl.ds(0, IDS_PER_CHUNK)]],
                buf, sem).wait()

        def consume_chunk(chunk, r_cur, buf):
            n = nrows_smem[chunk]

            def row_body(j, r):
                base = rstart_smem[r] - chunk * IDS_PER_CHUNK
                cntr = cnt_smem[r]

                def grp_body(g, carry):
                    accs = list(carry)
                    for u in range(LANES):
                        p = base + g * LANES + u
                        for h in range(H_REGS):
                            v = buf[p, pl.ds(h * LANES, LANES)]
                            accs[h] = accs[h] + v
                    return tuple(accs)

                def pos_body(p, carry):
                    accs = list(carry)
                    for h in range(H_REGS):
                        v = buf[base + p, pl.ds(h * LANES, LANES)]
                        accs[h] = accs[h] + v
                    return tuple(accs)

                zero = jnp.zeros((LANES,), jnp.float32)
                init = tuple(zero for _ in range(H_REGS))
                nfull = lax.shift_right_logical(cntr, 4)
                accs = lax.fori_loop(0, nfull, grp_body, init)
                accs = lax.fori_loop(nfull * LANES, cntr, pos_body, accs)
                rinv = 1.0 / jnp.full((LANES,), cntr.astype(jnp.float32),
                                      jnp.float32)
                for h in range(H_REGS):
                    pooled_v[r, pl.ds(h * LANES, LANES)] = accs[h] * rinv
                return r + 1

            return lax.fori_loop(0, n, row_body, r_cur)

        row_c, pos = compact_until(jnp.int32(0), jnp.int32(0), jnp.int32(0))
        gather_start(0, rows0, sem0)

        def chunk_pair(pair, carry):
            r_cur, row_c, pos = carry
            c0 = 2 * pair
            c1 = c0 + 1

            row_c, pos = compact_until(row_c, pos, c1)

            @pl.when(c1 * IDS_PER_CHUNK < pos)
            def _():
                gather_start(c1, rows1, sem1)

            gather_wait(rows0, sem0)
            r_cur = consume_chunk(c0, r_cur, rows0)

            row_c, pos = compact_until(row_c, pos, c0 + 2)

            @pl.when((c0 + 2) * IDS_PER_CHUNK < pos)
            def _():
                gather_start(c0 + 2, rows0, sem0)

            @pl.when(c1 * IDS_PER_CHUNK < pos)
            def _():
                gather_wait(rows1, sem1)

            r_cur = consume_chunk(c1, r_cur, rows1)
            return (r_cur, row_c, pos)

        def chunk_pair_guarded(pair, carry):
            return lax.cond(2 * pair * IDS_PER_CHUNK < carry[2], chunk_pair,
                            lambda _, c: c, pair, carry)

        lax.fori_loop(0, (MAX_CHUNKS + 1) // 2, chunk_pair_guarded,
                      (jnp.int32(0), row_c, pos))

        pltpu.sync_copy(pooled_v, out_hbm.at[pl.ds(row_base, B_PER_W)])

    return sc_pool


_sc_pool = _sc_pool_build()

_PROJ_BLOCK = 4096


def _tc_proj_body(x_ref, w_ref, o_ref):
    x = x_ref[...]
    w = w_ref[...]
    y = lax.dot_general(x, w, (((1,), (1,)), ((), ())),
                        preferred_element_type=jnp.float32)
    ss = jnp.sum(y * y, axis=1, keepdims=True)
    norm = jnp.maximum(jnp.sqrt(ss), 1e-8)
    o_ref[...] = y / norm


def _tc_proj(pooled, proj_weight):
    return pl.pallas_call(
        _tc_proj_body,
        out_shape=jax.ShapeDtypeStruct((B, OUT_DIM), jnp.float32),
        grid=(B // _PROJ_BLOCK,),
        in_specs=[
            pl.BlockSpec((_PROJ_BLOCK, HIDDEN), lambda i: (i, 0)),
            pl.BlockSpec((OUT_DIM, HIDDEN), lambda i: (0, 0)),
        ],
        out_specs=pl.BlockSpec((_PROJ_BLOCK, OUT_DIM), lambda i: (i, 0)),
    )(pooled, proj_weight)


def kernel(input_ids, attention_mask, embedding_table, proj_weight):
    pooled = _sc_pool(input_ids, attention_mask, embedding_table)
    return _tc_proj(pooled, proj_weight)


# speedup vs baseline: 1.1526x; 1.1526x over previous
import functools

import jax
import jax.numpy as jnp
from jax import lax
from jax.experimental import pallas as pl
from jax.experimental.pallas import tpu as pltpu
from jax.experimental.pallas import tpu_sc as plsc

B, L = 4096, 64
HIDDEN, OUT_DIM = 128, 256
LANES = 16
H_REGS = HIDDEN // LANES

NUM_CORES = 2
NUM_SUBCORES = 16
NW = NUM_CORES * NUM_SUBCORES
B_PER_W = B // NW

IDS_PER_CHUNK = 128
TOK_PER_W = B_PER_W * L
GROUPS = L // LANES
MAX_CHUNKS = B_PER_W + 1
CSTREAM = MAX_CHUNKS * IDS_PER_CHUNK + 2 * LANES


def _sc_pool_build():
    mesh = plsc.VectorSubcoreMesh(core_axis_name="c", subcore_axis_name="s")

    @functools.partial(
        pl.kernel,
        mesh=mesh,
        out_type=jax.ShapeDtypeStruct((B, HIDDEN), jnp.float32),
        scratch_types=[
            pltpu.VMEM((B_PER_W, L), jnp.int32),
            pltpu.VMEM((B_PER_W, L), jnp.float32),
            pltpu.VMEM((CSTREAM,), jnp.int32),
            pltpu.VMEM((IDS_PER_CHUNK, HIDDEN), jnp.float32),
            pltpu.VMEM((IDS_PER_CHUNK, HIDDEN), jnp.float32),
            pltpu.VMEM((IDS_PER_CHUNK, HIDDEN), jnp.float32),
            pltpu.VMEM((B_PER_W, HIDDEN), jnp.float32),
            pltpu.SMEM((B_PER_W,), jnp.int32),
            pltpu.SMEM((B_PER_W,), jnp.int32),
            pltpu.SMEM((MAX_CHUNKS + 2,), jnp.int32),
            pltpu.SemaphoreType.DMA,
            pltpu.SemaphoreType.DMA,
            pltpu.SemaphoreType.DMA,
        ],
        compiler_params=pltpu.CompilerParams(needs_layout_passes=False),
    )
    def sc_pool(ids_hbm, mask_hbm, table_hbm, out_hbm,
                idx_v, mask_v, cidx_v, rows0, rows1, rows2, pooled_v,
                cnt_smem, rstart_smem, nrows_smem, sem0, sem1, sem2):
        bufs = (rows0, rows1, rows2)
        sems = (sem0, sem1, sem2)
        wid = lax.axis_index("s") * NUM_CORES + lax.axis_index("c")
        row_base = wid * B_PER_W

        pltpu.sync_copy(ids_hbm.at[pl.ds(row_base, B_PER_W)], idx_v)
        pltpu.sync_copy(mask_hbm.at[pl.ds(row_base, B_PER_W)], mask_v)

        def zero_nrows(c, _):
            nrows_smem[c] = jnp.int32(0)
            return 0

        lax.fori_loop(0, MAX_CHUNKS + 2, zero_nrows, 0)

        pad16 = lax.iota(jnp.int32, LANES)
        pad_salt = (wid * jnp.int32(1009)) & jnp.int32(65535)

        def pad_to_bin(pos):
            pos_new = (pos + (IDS_PER_CHUNK - 1)) & ~(IDS_PER_CHUNK - 1)
            kpad = pos_new - pos

            def store_pad(j, _):
                p0 = pos + j * LANES
                base = (p0 * jnp.int32(37) + pad_salt) & jnp.int32(65535)
                cidx_v[pl.ds(p0, LANES)] = (
                    jnp.full((LANES,), base, jnp.int32) + pad16)
                return 0

            lax.fori_loop(0, (kpad + LANES - 1) >> 4, store_pad, 0)
            return pos_new

        def compact(row, pos):
            idgs, mgs, pcnts = [], [], []
            c = jnp.int32(0)
            for g in range(GROUPS):
                o = g * LANES
                idg = idx_v[row, pl.ds(o, LANES)]
                mg = mask_v[row, pl.ds(o, LANES)] > 0.0
                pc = plsc.all_reduce_population_count(mg)[0]
                idgs.append(idg)
                mgs.append(mg)
                pcnts.append(pc)
                c = c + pc
            rem = pos & (IDS_PER_CHUNK - 1)
            needs_new_bin = rem + c > IDS_PER_CHUNK
            pos = lax.cond(needs_new_bin, pad_to_bin, lambda p: p, pos)
            cnt_smem[row] = c
            rstart_smem[row] = pos
            b = lax.shift_right_logical(pos, 7)
            nrows_smem[b] = nrows_smem[b] + 1
            for g in range(GROUPS):
                plsc.store_compressed(cidx_v.at[pl.ds(pos, LANES)],
                                      idgs[g], mask=mgs[g])
                pos = pos + pcnts[g]
            return pos

        def compact_until(row_c, pos, target):
            def cond(st):
                r, p = st
                return jnp.logical_and(r < B_PER_W,
                                       p < (target + 1) * IDS_PER_CHUNK)

            def body(st):
                r, p = st
                return (r + 1, compact(r, p))

            row_c, pos = lax.while_loop(cond, body, (row_c, pos))
            pos = lax.cond(row_c == B_PER_W, pad_to_bin, lambda p: p, pos)
            return row_c, pos

        def gather_start(chunk, buf, sem):
            pltpu.make_async_copy(
                table_hbm.at[cidx_v.at[pl.ds(chunk * IDS_PER_CHUNK,
                                             IDS_PER_CHUNK)]],
                buf, sem).start()

        def gather_wait(buf, sem):
            pltpu.make_async_copy(
                table_hbm.at[cidx_v.at[pl.ds(0, IDS_PER_CHUNK)]],
                buf, sem).wait()

        def consume_chunk(chunk, r_cur, buf):
            n = nrows_smem[chunk]

            def row_body(j, r):
                base = rstart_smem[r] - chunk * IDS_PER_CHUNK
                cntr = cnt_smem[r]

                def grp_body(g, carry):
                    accs = list(carry)
                    for u in range(LANES):
                        p = base + g * LANES + u
                        for h in range(H_REGS):
                            v = buf[p, pl.ds(h * LANES, LANES)]
                            accs[h] = accs[h] + v
                    return tuple(accs)

                def pos_body(p, carry):
                    accs = list(carry)
                    for h in range(H_REGS):
                        v = buf[base + p, pl.ds(h * LANES, LANES)]
                        accs[h] = accs[h] + v
                    return tuple(accs)

                zero = jnp.zeros((LANES,), jnp.float32)
                init = tuple(zero for _ in range(H_REGS))
                nfull = lax.shift_right_logical(cntr, 4)
                accs = lax.fori_loop(0, nfull, grp_body, init)
                accs = lax.fori_loop(nfull * LANES, cntr, pos_body, accs)
                rinv = 1.0 / jnp.full((LANES,), cntr.astype(jnp.float32),
                                      jnp.float32)
                for h in range(H_REGS):
                    pooled_v[r, pl.ds(h * LANES, LANES)] = accs[h] * rinv
                return r + 1

            return lax.fori_loop(0, n, row_body, r_cur)

        NBUF = 3
        row_c, pos = compact_until(jnp.int32(0), jnp.int32(0),
                                   jnp.int32(NBUF - 2))
        gather_start(0, rows0, sem0)
        for k in range(1, NBUF - 1):
            @pl.when(k * IDS_PER_CHUNK < pos)
            def _(k=k):
                gather_start(k, bufs[k], sems[k])

        def chunk_quad(q, carry):
            r_cur, row_c, pos = carry
            for k in range(NBUF):
                c = NBUF * q + k
                nxt = c + (NBUF - 1)
                row_c, pos = compact_until(row_c, pos, nxt)

                @pl.when(nxt * IDS_PER_CHUNK < pos)
                def _(nxt=nxt, k=k):
                    gather_start(nxt, bufs[(k + NBUF - 1) % NBUF],
                                 sems[(k + NBUF - 1) % NBUF])

                @pl.when(c * IDS_PER_CHUNK < pos)
                def _(c=c, k=k):
                    gather_wait(bufs[k], sems[k])

                r_cur = consume_chunk(c, r_cur, bufs[k])
            return (r_cur, row_c, pos)

        def chunk_quad_guarded(q, carry):
            return lax.cond(NBUF * q * IDS_PER_CHUNK < carry[2], chunk_quad,
                            lambda _, c: c, q, carry)

        lax.fori_loop(0, (MAX_CHUNKS + NBUF - 1) // NBUF, chunk_quad_guarded,
                      (jnp.int32(0), row_c, pos))

        pltpu.sync_copy(pooled_v, out_hbm.at[pl.ds(row_base, B_PER_W)])

    return sc_pool


_sc_pool = _sc_pool_build()

_PROJ_BLOCK = 2048


def _tc_proj_body(x_ref, w_ref, o_ref):
    x = x_ref[...]
    w = w_ref[...]
    y = lax.dot_general(x, w, (((1,), (1,)), ((), ())),
                        preferred_element_type=jnp.float32)
    ss = jnp.sum(y * y, axis=1, keepdims=True)
    norm = jnp.maximum(jnp.sqrt(ss), 1e-8)
    o_ref[...] = y / norm


def _tc_proj(pooled, proj_weight):
    return pl.pallas_call(
        _tc_proj_body,
        out_shape=jax.ShapeDtypeStruct((B, OUT_DIM), jnp.float32),
        grid=(B // _PROJ_BLOCK,),
        in_specs=[
            pl.BlockSpec((_PROJ_BLOCK, HIDDEN), lambda i: (i, 0)),
            pl.BlockSpec((OUT_DIM, HIDDEN), lambda i: (0, 0)),
        ],
        out_specs=pl.BlockSpec((_PROJ_BLOCK, OUT_DIM), lambda i: (i, 0)),
    )(pooled, proj_weight)


def kernel(input_ids, attention_mask, embedding_table, proj_weight):
    pooled = _sc_pool(input_ids, attention_mask, embedding_table)
    return _tc_proj(pooled, proj_weight)
